# baseline (speedup 1.0000x reference)
import functools

import jax
import jax.numpy as jnp
from jax import lax
from jax.experimental import pallas as pl
from jax.experimental.pallas import tpu as pltpu
from jax.experimental.pallas import tpu_sc as plsc

N_NODES = 10000
D_FEAT = 128
HID = 64

NC = 2
NS = 16
NW = NC * NS
EB = 128
N_PAD = 10240
ROWS_PER_SUB = N_PAD // NS

_sc_params = pltpu.CompilerParams(use_tc_tiling_on_sc=False)


@functools.lru_cache(maxsize=1)
def _vec_mesh():
    return plsc.VectorSubcoreMesh(core_axis_name="c", subcore_axis_name="s")


def _agg_body(t_hbm, src_hbm, dst_hbm, z64_hbm, out_hbm,
              acc_sh, src_v, dst_v, rows0_v, rows1_v, sem0, sem1):
    cid = lax.axis_index("c")
    sid = lax.axis_index("s")
    nb = src_v.shape[0]
    pltpu.sync_copy(z64_hbm.at[pl.ds(sid * ROWS_PER_SUB, ROWS_PER_SUB)],
                    acc_sh.at[pl.ds(sid * ROWS_PER_SUB, ROWS_PER_SUB)])
    plsc.subcore_barrier()
    pltpu.sync_copy(src_hbm.at[cid, sid], src_v)
    pltpu.sync_copy(dst_hbm.at[cid, sid], dst_v)

    @pl.loop(0, nb, step=2)
    def _(j):
        g0 = pltpu.async_copy(t_hbm.at[src_v.at[j]], rows0_v, sem0)
        g1 = pltpu.async_copy(t_hbm.at[src_v.at[j + 1]], rows1_v, sem1)
        g0.wait()
        pltpu.sync_copy(rows0_v, acc_sh.at[dst_v.at[j]], add=True)
        g1.wait()
        pltpu.sync_copy(rows1_v, acc_sh.at[dst_v.at[j + 1]], add=True)

    plsc.subcore_barrier()
    pltpu.sync_copy(acc_sh.at[pl.ds(sid * ROWS_PER_SUB, ROWS_PER_SUB)],
                    out_hbm.at[cid, pl.ds(sid * ROWS_PER_SUB, ROWS_PER_SUB)])


def _agg_deg_body(t_hbm, src_hbm, dst_hbm, z64_hbm, z16_hbm, ones_hbm,
                  out_hbm, deg_hbm,
                  acc_sh, deg_sh, src_v, dst_v, rows0_v, rows1_v, ones_v,
                  sem0, sem1):
    cid = lax.axis_index("c")
    sid = lax.axis_index("s")
    nb = src_v.shape[0]
    pltpu.sync_copy(z64_hbm.at[pl.ds(sid * ROWS_PER_SUB, ROWS_PER_SUB)],
                    acc_sh.at[pl.ds(sid * ROWS_PER_SUB, ROWS_PER_SUB)])
    pltpu.sync_copy(z16_hbm.at[pl.ds(sid * ROWS_PER_SUB, ROWS_PER_SUB)],
                    deg_sh.at[pl.ds(sid * ROWS_PER_SUB, ROWS_PER_SUB)])
    pltpu.sync_copy(ones_hbm, ones_v)
    plsc.subcore_barrier()
    pltpu.sync_copy(src_hbm.at[cid, sid], src_v)
    pltpu.sync_copy(dst_hbm.at[cid, sid], dst_v)

    @pl.loop(0, nb, step=2)
    def _(j):
        g0 = pltpu.async_copy(t_hbm.at[src_v.at[j]], rows0_v, sem0)
        g1 = pltpu.async_copy(t_hbm.at[src_v.at[j + 1]], rows1_v, sem1)
        g0.wait()
        pltpu.sync_copy(rows0_v, acc_sh.at[dst_v.at[j]], add=True)
        pltpu.sync_copy(ones_v, deg_sh.at[dst_v.at[j]], add=True)
        g1.wait()
        pltpu.sync_copy(rows1_v, acc_sh.at[dst_v.at[j + 1]], add=True)
        pltpu.sync_copy(ones_v, deg_sh.at[dst_v.at[j + 1]], add=True)

    plsc.subcore_barrier()
    pltpu.sync_copy(acc_sh.at[pl.ds(sid * ROWS_PER_SUB, ROWS_PER_SUB)],
                    out_hbm.at[cid, pl.ds(sid * ROWS_PER_SUB, ROWS_PER_SUB)])
    pltpu.sync_copy(deg_sh.at[pl.ds(sid * ROWS_PER_SUB, ROWS_PER_SUB)],
                    deg_hbm.at[cid, pl.ds(sid * ROWS_PER_SUB, ROWS_PER_SUB)])


def _sc_aggregate(t, src_r, dst_r, zeros64, nb):
    k = pl.kernel(
        functools.partial(_agg_body),
        out_type=jax.ShapeDtypeStruct((NC, N_PAD, HID), jnp.float32),
        mesh=_vec_mesh(),
        compiler_params=_sc_params,
        scratch_types=[
            pltpu.VMEM_SHARED((N_PAD, HID), jnp.float32),
            pltpu.VMEM((nb, EB), jnp.int32),
            pltpu.VMEM((nb, EB), jnp.int32),
            pltpu.VMEM((EB, HID), jnp.float32),
            pltpu.VMEM((EB, HID), jnp.float32),
            pltpu.SemaphoreType.DMA,
            pltpu.SemaphoreType.DMA,
        ],
    )
    return k(t, src_r, dst_r, zeros64)


def _sc_aggregate_deg(t, src_r, dst_r, zeros64, zeros16, ones, nb):
    k = pl.kernel(
        functools.partial(_agg_deg_body),
        out_type=[jax.ShapeDtypeStruct((NC, N_PAD, HID), jnp.float32),
                  jax.ShapeDtypeStruct((NC, N_PAD, 16), jnp.float32)],
        mesh=_vec_mesh(),
        compiler_params=_sc_params,
        scratch_types=[
            pltpu.VMEM_SHARED((N_PAD, HID), jnp.float32),
            pltpu.VMEM_SHARED((N_PAD, 16), jnp.float32),
            pltpu.VMEM((nb, EB), jnp.int32),
            pltpu.VMEM((nb, EB), jnp.int32),
            pltpu.VMEM((EB, HID), jnp.float32),
            pltpu.VMEM((EB, HID), jnp.float32),
            pltpu.VMEM((EB, 16), jnp.float32),
            pltpu.SemaphoreType.DMA,
            pltpu.SemaphoreType.DMA,
        ],
    )
    return k(t, src_r, dst_r, zeros64, zeros16, ones)


def _enc_body(x_ref, qid_ref, rssi_ref, ap_ref, w1a_ref, w1b_ref, b1_ref,
              w2_ref, b2_ref, wn0x_ref, wn0z_ref, ws0x_ref, ws0z_ref, b0_ref,
              t0_ref, s0_ref):
    qid = qid_ref[...]
    nap = ap_ref.shape[0]
    oh = (qid == lax.broadcasted_iota(jnp.int32, (qid.shape[0], nap), 1))
    e = jnp.dot(oh.astype(jnp.float32), ap_ref[...],
                preferred_element_type=jnp.float32)
    h = jnp.dot(e, w1a_ref[...], preferred_element_type=jnp.float32)
    h = jax.nn.relu(h + rssi_ref[...] * w1b_ref[...] + b1_ref[...])
    h = jnp.dot(h, w2_ref[...], preferred_element_type=jnp.float32) + b2_ref[...]
    z_q = jnp.mean(h, axis=0, keepdims=True)
    ct = jnp.dot(z_q, wn0z_ref[...], preferred_element_type=jnp.float32)
    cs = jnp.dot(z_q, ws0z_ref[...], preferred_element_type=jnp.float32) + b0_ref[...]
    x = x_ref[...]
    t0_ref[...] = jnp.dot(x, wn0x_ref[...], preferred_element_type=jnp.float32) + ct
    s0_ref[...] = jnp.dot(x, ws0x_ref[...], preferred_element_type=jnp.float32) + cs


def _layer1_body(p_ref, deg_ref, s_prev_ref, wn_ref, ws_ref, b_ref,
                 t_ref, s_ref, invdeg_ref):
    deg = deg_ref[0, :N_NODES, 0:1] + deg_ref[1, :N_NODES, 0:1]
    invdeg = 1.0 / jnp.maximum(deg, 1.0)
    invdeg_ref[...] = invdeg
    a = (p_ref[0, :N_NODES, :] + p_ref[1, :N_NODES, :]) * invdeg
    h = jax.nn.relu(a + s_prev_ref[...])
    t_ref[...] = jnp.dot(h, wn_ref[...], preferred_element_type=jnp.float32)
    s_ref[...] = jnp.dot(h, ws_ref[...], preferred_element_type=jnp.float32) + b_ref[...]


def _layer2_body(p_ref, invdeg_ref, s_prev_ref, wn_ref, ws_ref, b_ref,
                 t_ref, s_ref):
    a = (p_ref[0, :N_NODES, :] + p_ref[1, :N_NODES, :]) * invdeg_ref[...]
    h = jax.nn.relu(a + s_prev_ref[...])
    t_ref[...] = jnp.dot(h, wn_ref[...], preferred_element_type=jnp.float32)
    s_ref[...] = jnp.dot(h, ws_ref[...], preferred_element_type=jnp.float32) + b_ref[...]


def _readout_body(p_ref, invdeg_ref, s_prev_ref, pos_ref,
                  w1_ref, b1_ref, w2_ref, b2_ref,
                  phat_ref, soft_ref):
    a = (p_ref[0, :N_NODES, :] + p_ref[1, :N_NODES, :]) * invdeg_ref[...]
    h = jax.nn.relu(a + s_prev_ref[...])
    u = jax.nn.relu(jnp.dot(h, w1_ref[...], preferred_element_type=jnp.float32)
                    + b1_ref[...])
    s = jnp.dot(u, w2_ref[...], preferred_element_type=jnp.float32) + b2_ref[...]
    m = jnp.max(s)
    ex = jnp.exp(s - m)
    soft = ex / jnp.sum(ex)
    soft_ref[...] = soft
    phat_ref[...] = jnp.sum(soft * pos_ref[...], axis=0, keepdims=True)


def _tc_call(body, out_shapes, *args):
    return pl.pallas_call(
        body,
        out_shape=out_shapes,
    )(*args)


def kernel(x, pos, edge_index, query_ap_ids, query_rssi, ap_emb,
           enc_W1, enc_b1, enc_W2, enc_b2,
           Wn0, Ws0, b0, Wn1, Ws1, b1, Wn2, Ws2, b2,
           sc_W1, sc_b1, sc_W2, sc_b2):
    N = x.shape[0]
    D = x.shape[1]
    E = edge_index.shape[1]
    n_q = query_ap_ids.shape[0]
    ap_emb_dim = ap_emb.shape[1]

    chunk = NW * EB * 2
    e_pad = ((E + chunk - 1) // chunk) * chunk
    nb = e_pad // (NW * EB)
    src = edge_index[0].astype(jnp.int32)
    dst = edge_index[1].astype(jnp.int32)
    pad = e_pad - E
    src_p = jnp.concatenate([src, jnp.zeros((pad,), jnp.int32)])
    dst_p = jnp.concatenate([dst, jnp.full((pad,), N, jnp.int32)])
    src_r = src_p.reshape(NC, NS, nb, EB)
    dst_r = dst_p.reshape(NC, NS, nb, EB)

    zeros64 = jnp.zeros((N_PAD, HID), jnp.float32)
    zeros16 = jnp.zeros((N_PAD, 16), jnp.float32)
    ones = jnp.ones((EB, 16), jnp.float32)

    t0, s0 = _tc_call(
        _enc_body,
        [jax.ShapeDtypeStruct((N, HID), jnp.float32),
         jax.ShapeDtypeStruct((N, HID), jnp.float32)],
        x, query_ap_ids.reshape(n_q, 1).astype(jnp.int32), query_rssi,
        ap_emb,
        enc_W1[:ap_emb_dim], enc_W1[ap_emb_dim:ap_emb_dim + 1],
        enc_b1.reshape(1, -1), enc_W2, enc_b2.reshape(1, -1),
        Wn0[:D], Wn0[D:], Ws0[:D], Ws0[D:], b0.reshape(1, -1))

    p0, deg = _sc_aggregate_deg(t0, src_r, dst_r, zeros64, zeros16, ones, nb)

    t1, s1, invdeg = _tc_call(
        _layer1_body,
        [jax.ShapeDtypeStruct((N, HID), jnp.float32),
         jax.ShapeDtypeStruct((N, HID), jnp.float32),
         jax.ShapeDtypeStruct((N, 1), jnp.float32)],
        p0, deg, s0, Wn1, Ws1, b1.reshape(1, -1))

    p1 = _sc_aggregate(t1, src_r, dst_r, zeros64, nb)

    t2, s2 = _tc_call(
        _layer2_body,
        [jax.ShapeDtypeStruct((N, HID), jnp.float32),
         jax.ShapeDtypeStruct((N, HID), jnp.float32)],
        p1, invdeg, s1, Wn2, Ws2, b2.reshape(1, -1))

    p2 = _sc_aggregate(t2, src_r, dst_r, zeros64, nb)

    phat, soft = _tc_call(
        _readout_body,
        [jax.ShapeDtypeStruct((1, 2), jnp.float32),
         jax.ShapeDtypeStruct((N, 1), jnp.float32)],
        p2, invdeg, s2, pos,
        sc_W1, sc_b1.reshape(1, -1), sc_W2, sc_b2.reshape(1, -1))

    return phat.reshape(2), soft.reshape(N)


# speedup vs baseline: 1.9434x; 1.9434x over previous
import functools

import jax
import jax.numpy as jnp
from jax import lax
from jax.experimental import pallas as pl
from jax.experimental.pallas import tpu as pltpu
from jax.experimental.pallas import tpu_sc as plsc

N_NODES = 10000
D_FEAT = 128
HID = 64

NC = 2
NS = 16
NW = NC * NS
EB = 128
N_PAD = 10240
ROWS_PER_SUB = N_PAD // NS

_sc_params = pltpu.CompilerParams(use_tc_tiling_on_sc=False)


@functools.lru_cache(maxsize=1)
def _vec_mesh():
    return plsc.VectorSubcoreMesh(core_axis_name="c", subcore_axis_name="s")


def _agg_body(t_hbm, src_hbm, dst_hbm, z64_hbm, out_hbm,
              acc_sh, t_sh, src_v, dst_v, rows_v, sem):
    cid = lax.axis_index("c")
    sid = lax.axis_index("s")
    nb = src_v.shape[0]
    n_t = t_sh.shape[0]
    pltpu.sync_copy(z64_hbm.at[pl.ds(sid * ROWS_PER_SUB, ROWS_PER_SUB)],
                    acc_sh.at[pl.ds(sid * ROWS_PER_SUB, ROWS_PER_SUB)])
    t_rows = n_t // NS
    pltpu.sync_copy(t_hbm.at[pl.ds(sid * t_rows, t_rows)],
                    t_sh.at[pl.ds(sid * t_rows, t_rows)])
    plsc.subcore_barrier()
    pltpu.sync_copy(src_hbm.at[cid, sid], src_v)
    pltpu.sync_copy(dst_hbm.at[cid, sid], dst_v)

    @pl.loop(0, nb)
    def _(j):
        pltpu.async_copy(t_sh.at[src_v.at[j]], rows_v, sem).wait()
        pltpu.sync_copy(rows_v, acc_sh.at[dst_v.at[j]], add=True)

    plsc.subcore_barrier()
    pltpu.sync_copy(acc_sh.at[pl.ds(sid * ROWS_PER_SUB, ROWS_PER_SUB)],
                    out_hbm.at[cid, pl.ds(sid * ROWS_PER_SUB, ROWS_PER_SUB)])


def _agg_deg_body(t_hbm, src_hbm, dst_hbm, z64_hbm, z16_hbm, ones_hbm,
                  out_hbm, deg_hbm,
                  acc_sh, deg_sh, t_sh, src_v, dst_v, rows_v, ones_v, sem):
    cid = lax.axis_index("c")
    sid = lax.axis_index("s")
    nb = src_v.shape[0]
    n_t = t_sh.shape[0]
    pltpu.sync_copy(z64_hbm.at[pl.ds(sid * ROWS_PER_SUB, ROWS_PER_SUB)],
                    acc_sh.at[pl.ds(sid * ROWS_PER_SUB, ROWS_PER_SUB)])
    pltpu.sync_copy(z16_hbm.at[pl.ds(sid * ROWS_PER_SUB, ROWS_PER_SUB)],
                    deg_sh.at[pl.ds(sid * ROWS_PER_SUB, ROWS_PER_SUB)])
    t_rows = n_t // NS
    pltpu.sync_copy(t_hbm.at[pl.ds(sid * t_rows, t_rows)],
                    t_sh.at[pl.ds(sid * t_rows, t_rows)])
    pltpu.sync_copy(ones_hbm, ones_v)
    plsc.subcore_barrier()
    pltpu.sync_copy(src_hbm.at[cid, sid], src_v)
    pltpu.sync_copy(dst_hbm.at[cid, sid], dst_v)

    @pl.loop(0, nb)
    def _(j):
        pltpu.async_copy(t_sh.at[src_v.at[j]], rows_v, sem).wait()
        pltpu.sync_copy(rows_v, acc_sh.at[dst_v.at[j]], add=True)
        pltpu.sync_copy(ones_v, deg_sh.at[dst_v.at[j]], add=True)

    plsc.subcore_barrier()
    pltpu.sync_copy(acc_sh.at[pl.ds(sid * ROWS_PER_SUB, ROWS_PER_SUB)],
                    out_hbm.at[cid, pl.ds(sid * ROWS_PER_SUB, ROWS_PER_SUB)])
    pltpu.sync_copy(deg_sh.at[pl.ds(sid * ROWS_PER_SUB, ROWS_PER_SUB)],
                    deg_hbm.at[cid, pl.ds(sid * ROWS_PER_SUB, ROWS_PER_SUB)])


def _sc_aggregate(t, src_r, dst_r, zeros64, nb):
    k = pl.kernel(
        functools.partial(_agg_body),
        out_type=jax.ShapeDtypeStruct((NC, N_PAD, HID), jnp.float32),
        mesh=_vec_mesh(),
        compiler_params=_sc_params,
        scratch_types=[
            pltpu.VMEM_SHARED((N_PAD, HID), jnp.float32),
            pltpu.VMEM_SHARED((N_NODES, HID), jnp.float32),
            pltpu.VMEM((nb, EB), jnp.int32),
            pltpu.VMEM((nb, EB), jnp.int32),
            pltpu.VMEM((EB, HID), jnp.float32),
            pltpu.SemaphoreType.DMA,
        ],
    )
    return k(t, src_r, dst_r, zeros64)


def _sc_aggregate_deg(t, src_r, dst_r, zeros64, zeros16, ones, nb):
    k = pl.kernel(
        functools.partial(_agg_deg_body),
        out_type=[jax.ShapeDtypeStruct((NC, N_PAD, HID), jnp.float32),
                  jax.ShapeDtypeStruct((NC, N_PAD, 16), jnp.float32)],
        mesh=_vec_mesh(),
        compiler_params=_sc_params,
        scratch_types=[
            pltpu.VMEM_SHARED((N_PAD, HID), jnp.float32),
            pltpu.VMEM_SHARED((N_PAD, 16), jnp.float32),
            pltpu.VMEM_SHARED((N_NODES, HID), jnp.float32),
            pltpu.VMEM((nb, EB), jnp.int32),
            pltpu.VMEM((nb, EB), jnp.int32),
            pltpu.VMEM((EB, HID), jnp.float32),
            pltpu.VMEM((EB, 16), jnp.float32),
            pltpu.SemaphoreType.DMA,
        ],
    )
    return k(t, src_r, dst_r, zeros64, zeros16, ones)


def _enc_body(x_ref, qid_ref, rssi_ref, ap_ref, w1a_ref, w1b_ref, b1_ref,
              w2_ref, b2_ref, wn0x_ref, wn0z_ref, ws0x_ref, ws0z_ref, b0_ref,
              t0_ref, s0_ref):
    qid = qid_ref[...]
    nap = ap_ref.shape[0]
    oh = (qid == lax.broadcasted_iota(jnp.int32, (qid.shape[0], nap), 1))
    e = jnp.dot(oh.astype(jnp.float32), ap_ref[...],
                preferred_element_type=jnp.float32)
    h = jnp.dot(e, w1a_ref[...], preferred_element_type=jnp.float32)
    h = jax.nn.relu(h + rssi_ref[...] * w1b_ref[...] + b1_ref[...])
    h = jnp.dot(h, w2_ref[...], preferred_element_type=jnp.float32) + b2_ref[...]
    z_q = jnp.mean(h, axis=0, keepdims=True)
    ct = jnp.dot(z_q, wn0z_ref[...], preferred_element_type=jnp.float32)
    cs = jnp.dot(z_q, ws0z_ref[...], preferred_element_type=jnp.float32) + b0_ref[...]
    x = x_ref[...]
    t0_ref[...] = jnp.dot(x, wn0x_ref[...], preferred_element_type=jnp.float32) + ct
    s0_ref[...] = jnp.dot(x, ws0x_ref[...], preferred_element_type=jnp.float32) + cs


def _layer1_body(p_ref, deg_ref, s_prev_ref, wn_ref, ws_ref, b_ref,
                 t_ref, s_ref, invdeg_ref):
    deg = deg_ref[0, :N_NODES, 0:1] + deg_ref[1, :N_NODES, 0:1]
    invdeg = 1.0 / jnp.maximum(deg, 1.0)
    invdeg_ref[...] = invdeg
    a = (p_ref[0, :N_NODES, :] + p_ref[1, :N_NODES, :]) * invdeg
    h = jax.nn.relu(a + s_prev_ref[...])
    t_ref[...] = jnp.dot(h, wn_ref[...], preferred_element_type=jnp.float32)
    s_ref[...] = jnp.dot(h, ws_ref[...], preferred_element_type=jnp.float32) + b_ref[...]


def _layer2_body(p_ref, invdeg_ref, s_prev_ref, wn_ref, ws_ref, b_ref,
                 t_ref, s_ref):
    a = (p_ref[0, :N_NODES, :] + p_ref[1, :N_NODES, :]) * invdeg_ref[...]
    h = jax.nn.relu(a + s_prev_ref[...])
    t_ref[...] = jnp.dot(h, wn_ref[...], preferred_element_type=jnp.float32)
    s_ref[...] = jnp.dot(h, ws_ref[...], preferred_element_type=jnp.float32) + b_ref[...]


def _readout_body(p_ref, invdeg_ref, s_prev_ref, pos_ref,
                  w1_ref, b1_ref, w2_ref, b2_ref,
                  phat_ref, soft_ref):
    a = (p_ref[0, :N_NODES, :] + p_ref[1, :N_NODES, :]) * invdeg_ref[...]
    h = jax.nn.relu(a + s_prev_ref[...])
    u = jax.nn.relu(jnp.dot(h, w1_ref[...], preferred_element_type=jnp.float32)
                    + b1_ref[...])
    s = jnp.dot(u, w2_ref[...], preferred_element_type=jnp.float32) + b2_ref[...]
    m = jnp.max(s)
    ex = jnp.exp(s - m)
    soft = ex / jnp.sum(ex)
    soft_ref[...] = soft
    phat_ref[...] = jnp.sum(soft * pos_ref[...], axis=0, keepdims=True)


def _tc_call(body, out_shapes, *args):
    return pl.pallas_call(
        body,
        out_shape=out_shapes,
    )(*args)


def kernel(x, pos, edge_index, query_ap_ids, query_rssi, ap_emb,
           enc_W1, enc_b1, enc_W2, enc_b2,
           Wn0, Ws0, b0, Wn1, Ws1, b1, Wn2, Ws2, b2,
           sc_W1, sc_b1, sc_W2, sc_b2):
    N = x.shape[0]
    D = x.shape[1]
    E = edge_index.shape[1]
    n_q = query_ap_ids.shape[0]
    ap_emb_dim = ap_emb.shape[1]

    chunk = NW * EB * 2
    e_pad = ((E + chunk - 1) // chunk) * chunk
    nb = e_pad // (NW * EB)
    src = edge_index[0].astype(jnp.int32)
    dst = edge_index[1].astype(jnp.int32)
    pad = e_pad - E
    src_p = jnp.concatenate([src, jnp.zeros((pad,), jnp.int32)])
    dst_p = jnp.concatenate([dst, jnp.full((pad,), N, jnp.int32)])
    src_r = src_p.reshape(NC, NS, nb, EB)
    dst_r = dst_p.reshape(NC, NS, nb, EB)

    zeros64 = jnp.zeros((N_PAD, HID), jnp.float32)
    zeros16 = jnp.zeros((N_PAD, 16), jnp.float32)
    ones = jnp.ones((EB, 16), jnp.float32)

    t0, s0 = _tc_call(
        _enc_body,
        [jax.ShapeDtypeStruct((N, HID), jnp.float32),
         jax.ShapeDtypeStruct((N, HID), jnp.float32)],
        x, query_ap_ids.reshape(n_q, 1).astype(jnp.int32), query_rssi,
        ap_emb,
        enc_W1[:ap_emb_dim], enc_W1[ap_emb_dim:ap_emb_dim + 1],
        enc_b1.reshape(1, -1), enc_W2, enc_b2.reshape(1, -1),
        Wn0[:D], Wn0[D:], Ws0[:D], Ws0[D:], b0.reshape(1, -1))

    p0, deg = _sc_aggregate_deg(t0, src_r, dst_r, zeros64, zeros16, ones, nb)

    t1, s1, invdeg = _tc_call(
        _layer1_body,
        [jax.ShapeDtypeStruct((N, HID), jnp.float32),
         jax.ShapeDtypeStruct((N, HID), jnp.float32),
         jax.ShapeDtypeStruct((N, 1), jnp.float32)],
        p0, deg, s0, Wn1, Ws1, b1.reshape(1, -1))

    p1 = _sc_aggregate(t1, src_r, dst_r, zeros64, nb)

    t2, s2 = _tc_call(
        _layer2_body,
        [jax.ShapeDtypeStruct((N, HID), jnp.float32),
         jax.ShapeDtypeStruct((N, HID), jnp.float32)],
        p1, invdeg, s1, Wn2, Ws2, b2.reshape(1, -1))

    p2 = _sc_aggregate(t2, src_r, dst_r, zeros64, nb)

    phat, soft = _tc_call(
        _readout_body,
        [jax.ShapeDtypeStruct((1, 2), jnp.float32),
         jax.ShapeDtypeStruct((N, 1), jnp.float32)],
        p2, invdeg, s2, pos,
        sc_W1, sc_b1.reshape(1, -1), sc_W2, sc_b2.reshape(1, -1))

    return phat.reshape(2), soft.reshape(N)
